# baseline (speedup 1.0000x reference)
import functools

import jax
import jax.numpy as jnp
from jax.experimental import pallas as pl
from jax.experimental.pallas import tpu as pltpu
from jax.experimental.pallas import tpu_sc as plsc

_D = 128

_smesh = plsc.ScalarSubcoreMesh(axis_name="c", num_cores=1)


@functools.partial(
    pl.kernel,
    out_type=jax.ShapeDtypeStruct((2, _D), jnp.float32),
    mesh=_smesh,
    scratch_types=[
        pltpu.SMEM((8, _D), jnp.float32),
        pltpu.SMEM((2, _D), jnp.float32),
    ],
)
def _bag_scs(x_hbm, out_hbm, rows_s, out_s):
    pltpu.sync_copy(x_hbm.at[pl.ds(0, 8)], rows_s)
    for i in range(_D):
        r1 = rows_s[1, i]
        r2 = rows_s[2, i]
        r3 = rows_s[3, i]
        r4 = rows_s[4, i]
        t = r1 + r3
        out_s[0, i] = t + r2
        out_s[1, i] = t + r4
    pltpu.sync_copy(out_s, out_hbm)


def kernel(x):
    return _bag_scs(x)


# speedup vs baseline: 14.0778x; 14.0778x over previous
import jax
import jax.numpy as jnp
from jax.experimental import pallas as pl


def _bag_sum_body(x_ref, o_ref):
    g = x_ref[...]
    t = g[1:2] + g[3:4]
    o_ref[0:1, :] = t + g[2:3]
    o_ref[1:2, :] = t + g[4:5]


def kernel(x):
    return pl.pallas_call(
        _bag_sum_body,
        grid=(1,),
        in_specs=[pl.BlockSpec((8, 128), lambda i: (0, 0))],
        out_specs=pl.BlockSpec((2, 128), lambda i: (0, 0)),
        out_shape=jax.ShapeDtypeStruct((2, 128), jnp.float32),
    )(x)
